# baseline (device time: 33272 ns/iter reference)
import jax
import jax.numpy as jnp
from jax import lax
from jax.experimental import pallas as pl
from jax.experimental.pallas import tpu as pltpu

N_CHUNKS = 8


def kernel(x):
    m, n = x.shape
    n_half = n // 2
    cm = m // N_CHUNKS

    def body(x_ref, out_ref, send_buf, recv_buf, send_sems, recv_sems):
        my_x = lax.axis_index("x")
        my_y = lax.axis_index("y")
        my_z = lax.axis_index("z")
        peer = (1 - my_x, my_y, my_z)
        src_col = (1 - my_x) * n_half

        barrier_sem = pltpu.get_barrier_semaphore()
        pl.semaphore_signal(
            barrier_sem, inc=1, device_id=peer,
            device_id_type=pl.DeviceIdType.MESH,
        )
        pl.semaphore_wait(barrier_sem, 1)

        rdmas = []
        for i in range(N_CHUNKS):
            slot = i % 2
            if i >= 2:
                rdmas[i - 2].wait_send()
            chunk = x_ref[pl.ds(i * cm, cm), pl.ds(src_col, n_half)]
            send_buf[slot] = chunk.astype(jnp.bfloat16)
            rdma = pltpu.make_async_remote_copy(
                src_ref=send_buf.at[slot],
                dst_ref=recv_buf.at[i],
                send_sem=send_sems.at[slot],
                recv_sem=recv_sems.at[i],
                device_id=peer,
                device_id_type=pl.DeviceIdType.MESH,
            )
            rdma.start()
            rdmas.append(rdma)

        out_ref[pl.ds(my_x * m, m), :] = x_ref[:, pl.ds(my_x * n_half, n_half)]

        peer_row0 = (1 - my_x) * m
        for i in range(N_CHUNKS):
            rdmas[i].wait_recv()
            out_ref[pl.ds(peer_row0 + i * cm, cm), :] = recv_buf[i].astype(
                jnp.float32
            )

        rdmas[N_CHUNKS - 2].wait_send()
        rdmas[N_CHUNKS - 1].wait_send()

    return pl.pallas_call(
        body,
        out_shape=jax.ShapeDtypeStruct((2 * m, n_half), x.dtype),
        in_specs=[pl.BlockSpec(memory_space=pltpu.VMEM)],
        out_specs=pl.BlockSpec(memory_space=pltpu.VMEM),
        scratch_shapes=[
            pltpu.VMEM((2, cm, n_half), jnp.bfloat16),
            pltpu.VMEM((N_CHUNKS, cm, n_half), jnp.bfloat16),
            pltpu.SemaphoreType.DMA((2,)),
            pltpu.SemaphoreType.DMA((N_CHUNKS,)),
        ],
        compiler_params=pltpu.CompilerParams(collective_id=0),
    )(x)
